# baseline (speedup 1.0000x reference)
import functools

import jax
import jax.numpy as jnp
from jax import lax
from jax.experimental import pallas as pl
from jax.experimental.pallas import tpu as pltpu
from jax.experimental.pallas import tpu_sc as plsc

_ACT_SCALE = 1.0
_NC = 2
_NS = 16
_CHUNK = 128


def _mlp_dist_body(x_ref, W1_ref, b1_ref, W2_ref, b2_ref, Wfc_ref, bfc_ref,
                   cb_ref, cbh_ref, z_ref, idx_ref, gidx_ref, *, K):
    x = x_ref[...] / _ACT_SCALE
    h = jnp.maximum(
        jnp.dot(x, W1_ref[...], preferred_element_type=jnp.float32) + b1_ref[...], 0.0)
    h = jnp.maximum(
        jnp.dot(h, W2_ref[...], preferred_element_type=jnp.float32) + b2_ref[...], 0.0)
    z = jnp.dot(h, Wfc_ref[...], preferred_element_type=jnp.float32) + bfc_ref[...]
    z_ref[...] = z
    _dist_argmin(z, cb_ref, cbh_ref, idx_ref, gidx_ref, g=0, K=K)


def _dist_argmin(residual, cb_ref, cbh_ref, idx_ref, gidx_ref, *, g, K):
    rn = jnp.sum(residual * residual, axis=-1, keepdims=True)
    zc = jax.lax.dot_general(residual.astype(jnp.bfloat16), cbh_ref[...],
                             (((1,), (1,)), ((), ())),
                             preferred_element_type=jnp.float32)
    cb32 = cb_ref[...]
    cn = jnp.sum(cb32 * cb32, axis=-1)[None, :]
    dist = (rn - 2.0 * zc) + cn
    m = jnp.min(dist, axis=-1, keepdims=True)
    iota = jax.lax.broadcasted_iota(jnp.int32, dist.shape, 1)
    idx = jnp.min(jnp.where(dist == m, iota, K), axis=-1)
    idx_ref[...] = idx
    gidx_ref[...] = idx + g * K


def _sub_dist_body(r_ref, q_ref, cb_ref, cbh_ref,
                   rout_ref, idx_ref, gidx_ref, *, g, K):
    residual = r_ref[...] - q_ref[...]
    rout_ref[...] = residual
    _dist_argmin(residual, cb_ref, cbh_ref, idx_ref, gidx_ref, g=g, K=K)


def _final_body(z_ref, r_ref, q_ref, out_ref):
    z = z_ref[...]
    r4 = r_ref[...] - q_ref[...]
    qout = z - r4
    out_ref[...] = z + (qout - z)


def _sc_gather_body(table_hbm, gidx_hbm, out_hbm, idx_v, rows_v, sem):
    wid = lax.axis_index("s") * _NC + lax.axis_index("c")
    n_chunks = 16384 // (_NC * _NS) // _CHUNK
    for c in range(n_chunks):
        base = wid * (n_chunks * _CHUNK) + c * _CHUNK
        pltpu.sync_copy(gidx_hbm.at[pl.ds(base, _CHUNK)], idx_v)
        pltpu.async_copy(table_hbm.at[idx_v], rows_v, sem).wait()
        pltpu.sync_copy(rows_v, out_hbm.at[pl.ds(base, _CHUNK)])


def _sc_gather(cb_flat, gidx, B, D):
    mesh = plsc.VectorSubcoreMesh(core_axis_name="c", subcore_axis_name="s")
    return pl.kernel(
        _sc_gather_body,
        mesh=mesh,
        out_type=jax.ShapeDtypeStruct((B, D), jnp.float32),
        scratch_types=[
            pltpu.VMEM((_CHUNK,), jnp.int32),
            pltpu.VMEM((_CHUNK, D), jnp.float32),
            pltpu.SemaphoreType.DMA,
        ],
    )(cb_flat, gidx)


def kernel(state, W1, b1, W2, b2, Wfc, bfc, codebooks):
    B = state.shape[0]
    x = state.reshape(B, -1)
    in_dim = x.shape[1]
    D = Wfc.shape[1]
    G, K, _ = codebooks.shape

    cbh = codebooks.astype(jnp.bfloat16)
    cb_flat = codebooks.reshape(G * K, D)

    bm = min(1024, B)
    grid = (B // bm,)
    full = lambda shape: pl.BlockSpec(shape, lambda i: tuple(0 for _ in shape))
    row_spec = pl.BlockSpec((bm, D), lambda i: (i, 0))
    vec_spec = pl.BlockSpec((bm,), lambda i: (i,))

    z, idx0, gidx0 = pl.pallas_call(
        functools.partial(_mlp_dist_body, K=K),
        grid=grid,
        in_specs=[
            pl.BlockSpec((bm, in_dim), lambda i: (i, 0)),
            full(W1.shape), full(b1.shape), full(W2.shape), full(b2.shape),
            full(Wfc.shape), full(bfc.shape),
            full((K, D)), full((K, D)),
        ],
        out_specs=[row_spec, vec_spec, vec_spec],
        out_shape=[
            jax.ShapeDtypeStruct((B, D), jnp.float32),
            jax.ShapeDtypeStruct((B,), jnp.int32),
            jax.ShapeDtypeStruct((B,), jnp.int32),
        ],
    )(x, W1, b1, W2, b2, Wfc, bfc, codebooks[0], cbh[0])

    codes = [idx0]
    r = z
    gidx = gidx0
    q = None
    for g in range(1, G):
        q = _sc_gather(cb_flat, gidx, B, D)
        r, idxg, gidx = pl.pallas_call(
            functools.partial(_sub_dist_body, g=g, K=K),
            grid=grid,
            in_specs=[row_spec, row_spec, full((K, D)), full((K, D))],
            out_specs=[row_spec, vec_spec, vec_spec],
            out_shape=[
                jax.ShapeDtypeStruct((B, D), jnp.float32),
                jax.ShapeDtypeStruct((B,), jnp.int32),
                jax.ShapeDtypeStruct((B,), jnp.int32),
            ],
        )(r, q, codebooks[g], cbh[g])
        codes.append(idxg)

    q = _sc_gather(cb_flat, gidx, B, D)
    out_vq = pl.pallas_call(
        _final_body,
        grid=grid,
        in_specs=[row_spec, row_spec, row_spec],
        out_specs=row_spec,
        out_shape=jax.ShapeDtypeStruct((B, D), jnp.float32),
    )(z, r, q)

    vq_code = jnp.stack(codes, axis=-1)
    return out_vq, vq_code


# speedup vs baseline: 1.0030x; 1.0030x over previous
import functools

import jax
import jax.numpy as jnp
from jax import lax
from jax.experimental import pallas as pl
from jax.experimental.pallas import tpu as pltpu
from jax.experimental.pallas import tpu_sc as plsc

_ACT_SCALE = 1.0
_NC = 2
_NS = 16
_CHUNK = 128


def _mlp_dist_body(x_ref, W1_ref, b1_ref, W2_ref, b2_ref, Wfc_ref, bfc_ref,
                   cb_ref, cbh_ref, z_ref, idx_ref, gidx_ref, *, K):
    x = x_ref[...] / _ACT_SCALE
    h = jnp.maximum(
        jnp.dot(x, W1_ref[...], preferred_element_type=jnp.float32) + b1_ref[...], 0.0)
    h = jnp.maximum(
        jnp.dot(h, W2_ref[...], preferred_element_type=jnp.float32) + b2_ref[...], 0.0)
    z = jnp.dot(h, Wfc_ref[...], preferred_element_type=jnp.float32) + bfc_ref[...]
    z_ref[...] = z
    _dist_argmin(z, cb_ref, cbh_ref, idx_ref, gidx_ref, g=0, K=K)


def _dist_argmin(residual, cb_ref, cbh_ref, idx_ref, gidx_ref, *, g, K):
    rn = jnp.sum(residual * residual, axis=-1, keepdims=True)
    zc = jax.lax.dot_general(residual.astype(jnp.bfloat16), cbh_ref[...],
                             (((1,), (1,)), ((), ())),
                             preferred_element_type=jnp.float32)
    cb32 = cb_ref[...]
    cn = jnp.sum(cb32 * cb32, axis=-1)[None, :]
    dist = (rn - 2.0 * zc) + cn
    m = jnp.min(dist, axis=-1, keepdims=True)
    iota = jax.lax.broadcasted_iota(jnp.int32, dist.shape, 1)
    idx = jnp.min(jnp.where(dist == m, iota, K), axis=-1)
    idx_ref[...] = idx
    gidx_ref[...] = idx + g * K


def _sub_dist_body(r_ref, q_ref, cb_ref, cbh_ref,
                   rout_ref, idx_ref, gidx_ref, *, g, K):
    residual = r_ref[...] - q_ref[...]
    rout_ref[...] = residual
    _dist_argmin(residual, cb_ref, cbh_ref, idx_ref, gidx_ref, g=g, K=K)


def _final_body(z_ref, r_ref, q_ref, out_ref):
    z = z_ref[...]
    r4 = r_ref[...] - q_ref[...]
    qout = z - r4
    out_ref[...] = z + (qout - z)


def _sc_gather_body(table_hbm, gidx_hbm, out_hbm,
                    i0, i1, i2, i3, rows0, rows1, sem0, sem1):
    wid = lax.axis_index("s") * _NC + lax.axis_index("c")
    n_chunks = 16384 // (_NC * _NS) // _CHUNK
    base = wid * (n_chunks * _CHUNK)
    idx_bufs = (i0, i1, i2, i3)
    for c in range(n_chunks):
        pltpu.sync_copy(gidx_hbm.at[pl.ds(base + c * _CHUNK, _CHUNK)], idx_bufs[c])
    rows = (rows0, rows1)
    sems = (sem0, sem1)
    copies = [None, None]
    copies[0] = pltpu.async_copy(table_hbm.at[idx_bufs[0]], rows[0], sems[0])
    for c in range(n_chunks):
        b = c % 2
        if c + 1 < n_chunks:
            nb = (c + 1) % 2
            copies[nb] = pltpu.async_copy(table_hbm.at[idx_bufs[c + 1]], rows[nb], sems[nb])
        copies[b].wait()
        pltpu.sync_copy(rows[b], out_hbm.at[pl.ds(base + c * _CHUNK, _CHUNK)])


def _sc_gather(cb_flat, gidx, B, D):
    mesh = plsc.VectorSubcoreMesh(core_axis_name="c", subcore_axis_name="s")
    return pl.kernel(
        _sc_gather_body,
        mesh=mesh,
        out_type=jax.ShapeDtypeStruct((B, D), jnp.float32),
        scratch_types=[
            pltpu.VMEM((_CHUNK,), jnp.int32),
            pltpu.VMEM((_CHUNK,), jnp.int32),
            pltpu.VMEM((_CHUNK,), jnp.int32),
            pltpu.VMEM((_CHUNK,), jnp.int32),
            pltpu.VMEM((_CHUNK, D), jnp.float32),
            pltpu.VMEM((_CHUNK, D), jnp.float32),
            pltpu.SemaphoreType.DMA,
            pltpu.SemaphoreType.DMA,
        ],
    )(cb_flat, gidx)


def kernel(state, W1, b1, W2, b2, Wfc, bfc, codebooks):
    B = state.shape[0]
    x = state.reshape(B, -1)
    in_dim = x.shape[1]
    D = Wfc.shape[1]
    G, K, _ = codebooks.shape

    cbh = codebooks.astype(jnp.bfloat16)
    cb_flat = codebooks.reshape(G * K, D)

    bm = min(1024, B)
    grid = (B // bm,)
    full = lambda shape: pl.BlockSpec(shape, lambda i: tuple(0 for _ in shape))
    row_spec = pl.BlockSpec((bm, D), lambda i: (i, 0))
    vec_spec = pl.BlockSpec((bm,), lambda i: (i,))

    z, idx0, gidx0 = pl.pallas_call(
        functools.partial(_mlp_dist_body, K=K),
        grid=grid,
        in_specs=[
            pl.BlockSpec((bm, in_dim), lambda i: (i, 0)),
            full(W1.shape), full(b1.shape), full(W2.shape), full(b2.shape),
            full(Wfc.shape), full(bfc.shape),
            full((K, D)), full((K, D)),
        ],
        out_specs=[row_spec, vec_spec, vec_spec],
        out_shape=[
            jax.ShapeDtypeStruct((B, D), jnp.float32),
            jax.ShapeDtypeStruct((B,), jnp.int32),
            jax.ShapeDtypeStruct((B,), jnp.int32),
        ],
    )(x, W1, b1, W2, b2, Wfc, bfc, codebooks[0], cbh[0])

    codes = [idx0]
    r = z
    gidx = gidx0
    q = None
    for g in range(1, G):
        q = _sc_gather(cb_flat, gidx, B, D)
        r, idxg, gidx = pl.pallas_call(
            functools.partial(_sub_dist_body, g=g, K=K),
            grid=grid,
            in_specs=[row_spec, row_spec, full((K, D)), full((K, D))],
            out_specs=[row_spec, vec_spec, vec_spec],
            out_shape=[
                jax.ShapeDtypeStruct((B, D), jnp.float32),
                jax.ShapeDtypeStruct((B,), jnp.int32),
                jax.ShapeDtypeStruct((B,), jnp.int32),
            ],
        )(r, q, codebooks[g], cbh[g])
        codes.append(idxg)

    q = _sc_gather(cb_flat, gidx, B, D)
    out_vq = pl.pallas_call(
        _final_body,
        grid=grid,
        in_specs=[row_spec, row_spec, row_spec],
        out_specs=row_spec,
        out_shape=jax.ShapeDtypeStruct((B, D), jnp.float32),
    )(z, r, q)

    vq_code = jnp.stack(codes, axis=-1)
    return out_vq, vq_code


# speedup vs baseline: 1.1678x; 1.1643x over previous
import functools

import jax
import jax.numpy as jnp
from jax.experimental import pallas as pl

_ACT_SCALE = 1.0


def _vq_body(x_ref, W1_ref, b1_ref, W2_ref, b2_ref, Wfc_ref, bfc_ref,
             cb_ref, cbh_ref,
             out_ref, c0_ref, c1_ref, c2_ref, c3_ref, *, G, K):
    x = x_ref[...] / _ACT_SCALE
    h = jnp.maximum(
        jnp.dot(x, W1_ref[...], preferred_element_type=jnp.float32) + b1_ref[...], 0.0)
    h = jnp.maximum(
        jnp.dot(h, W2_ref[...], preferred_element_type=jnp.float32) + b2_ref[...], 0.0)
    z = jnp.dot(h, Wfc_ref[...], preferred_element_type=jnp.float32) + bfc_ref[...]

    code_refs = (c0_ref, c1_ref, c2_ref, c3_ref)
    residual = z
    qout = jnp.zeros_like(z)
    for g in range(G):
        rn = jnp.sum(residual * residual, axis=-1, keepdims=True)
        zc = jax.lax.dot_general(residual.astype(jnp.bfloat16), cbh_ref[g],
                                 (((1,), (1,)), ((), ())),
                                 preferred_element_type=jnp.float32)
        cb32 = cb_ref[g]
        cn = jnp.sum(cb32 * cb32, axis=-1)[None, :]
        dist = (rn - 2.0 * zc) + cn
        m = jnp.min(dist, axis=-1, keepdims=True)
        iota = jax.lax.broadcasted_iota(jnp.int32, dist.shape, 1)
        idx = jnp.min(jnp.where(dist == m, iota, K), axis=-1)
        oh = (iota == idx[:, None]).astype(jnp.float32)
        q = jax.lax.dot_general(oh, cb32, (((1,), (0,)), ((), ())),
                                preferred_element_type=jnp.float32,
                                precision=jax.lax.Precision.HIGHEST)
        residual = residual - q
        qout = qout + q
        code_refs[g][...] = idx
    out_ref[...] = z + (qout - z)


def kernel(state, W1, b1, W2, b2, Wfc, bfc, codebooks):
    B = state.shape[0]
    x = state.reshape(B, -1)
    in_dim = x.shape[1]
    D = Wfc.shape[1]
    G, K, _ = codebooks.shape

    cbh = codebooks.astype(jnp.bfloat16)

    bm = min(2048, B)
    grid = (B // bm,)

    full = lambda shape: pl.BlockSpec(shape, lambda i: tuple(0 for _ in shape))
    out_vq, c0, c1, c2, c3 = pl.pallas_call(
        functools.partial(_vq_body, G=G, K=K),
        grid=grid,
        in_specs=[
            pl.BlockSpec((bm, in_dim), lambda i: (i, 0)),
            full(W1.shape), full(b1.shape), full(W2.shape), full(b2.shape),
            full(Wfc.shape), full(bfc.shape),
            full(codebooks.shape), full(cbh.shape),
        ],
        out_specs=[
            pl.BlockSpec((bm, D), lambda i: (i, 0)),
            pl.BlockSpec((bm,), lambda i: (i,)),
            pl.BlockSpec((bm,), lambda i: (i,)),
            pl.BlockSpec((bm,), lambda i: (i,)),
            pl.BlockSpec((bm,), lambda i: (i,)),
        ],
        out_shape=[
            jax.ShapeDtypeStruct((B, D), jnp.float32),
            jax.ShapeDtypeStruct((B,), jnp.int32),
            jax.ShapeDtypeStruct((B,), jnp.int32),
            jax.ShapeDtypeStruct((B,), jnp.int32),
            jax.ShapeDtypeStruct((B,), jnp.int32),
        ],
    )(x, W1, b1, W2, b2, Wfc, bfc, codebooks, cbh)
    vq_code = jnp.stack([c0, c1, c2, c3], axis=-1)
    return out_vq, vq_code


# speedup vs baseline: 1.2738x; 1.0908x over previous
import functools

import jax
import jax.numpy as jnp
from jax.experimental import pallas as pl

_ACT_SCALE = 1.0


def _vq_body(x_ref, W1_ref, b1_ref, W2_ref, b2_ref, Wfc_ref, bfc_ref,
             cb_ref, cbh_ref,
             out_ref, c0_ref, c1_ref, c2_ref, c3_ref, *, G, K):
    x = x_ref[...] / _ACT_SCALE
    h = jnp.maximum(
        jnp.dot(x, W1_ref[...], preferred_element_type=jnp.float32) + b1_ref[...], 0.0)
    h = jnp.maximum(
        jnp.dot(h, W2_ref[...], preferred_element_type=jnp.float32) + b2_ref[...], 0.0)
    z = jnp.dot(h, Wfc_ref[...], preferred_element_type=jnp.float32) + bfc_ref[...]

    code_refs = (c0_ref, c1_ref, c2_ref, c3_ref)
    residual = z
    qout = jnp.zeros_like(z)
    for g in range(G):
        rn = jnp.sum(residual * residual, axis=-1, keepdims=True)
        zc = jax.lax.dot_general(residual.astype(jnp.bfloat16), cbh_ref[g],
                                 (((1,), (1,)), ((), ())),
                                 preferred_element_type=jnp.float32)
        cb32 = cb_ref[g]
        cn = jnp.sum(cb32 * cb32, axis=-1)[None, :]
        dist = (rn - 2.0 * zc) + cn
        m = jnp.min(dist, axis=-1, keepdims=True)
        iota = jax.lax.broadcasted_iota(jnp.int32, dist.shape, 1)
        idx = jnp.min(jnp.where(dist == m, iota, K), axis=-1)
        oh = (iota == idx[:, None]).astype(jnp.float32)
        q = jax.lax.dot_general(oh, cb32, (((1,), (0,)), ((), ())),
                                preferred_element_type=jnp.float32,
                                precision=jax.lax.Precision.HIGHEST)
        residual = residual - q
        qout = qout + q
        code_refs[g][...] = idx
    out_ref[...] = z + (qout - z)


def kernel(state, W1, b1, W2, b2, Wfc, bfc, codebooks):
    B = state.shape[0]
    x = state.reshape(B, -1)
    in_dim = x.shape[1]
    D = Wfc.shape[1]
    G, K, _ = codebooks.shape

    cbh = codebooks.astype(jnp.bfloat16)

    bm = min(512, B)
    grid = (B // bm,)

    full = lambda shape: pl.BlockSpec(shape, lambda i: tuple(0 for _ in shape))
    out_vq, c0, c1, c2, c3 = pl.pallas_call(
        functools.partial(_vq_body, G=G, K=K),
        grid=grid,
        in_specs=[
            pl.BlockSpec((bm, in_dim), lambda i: (i, 0)),
            full(W1.shape), full(b1.shape), full(W2.shape), full(b2.shape),
            full(Wfc.shape), full(bfc.shape),
            full(codebooks.shape), full(cbh.shape),
        ],
        out_specs=[
            pl.BlockSpec((bm, D), lambda i: (i, 0)),
            pl.BlockSpec((bm,), lambda i: (i,)),
            pl.BlockSpec((bm,), lambda i: (i,)),
            pl.BlockSpec((bm,), lambda i: (i,)),
            pl.BlockSpec((bm,), lambda i: (i,)),
        ],
        out_shape=[
            jax.ShapeDtypeStruct((B, D), jnp.float32),
            jax.ShapeDtypeStruct((B,), jnp.int32),
            jax.ShapeDtypeStruct((B,), jnp.int32),
            jax.ShapeDtypeStruct((B,), jnp.int32),
            jax.ShapeDtypeStruct((B,), jnp.int32),
        ],
    )(x, W1, b1, W2, b2, Wfc, bfc, codebooks, cbh)
    vq_code = jnp.stack([c0, c1, c2, c3], axis=-1)
    return out_vq, vq_code


# speedup vs baseline: 1.3427x; 1.0541x over previous
import functools

import jax
import jax.numpy as jnp
from jax.experimental import pallas as pl

_ACT_SCALE = 1.0


def _vq_body(x_ref, W1_ref, b1_ref, W2_ref, b2_ref, Wfc_ref, bfc_ref,
             cb_ref, cbh_ref,
             out_ref, c0_ref, c1_ref, c2_ref, c3_ref, *, G, K):
    x = x_ref[...] / _ACT_SCALE
    h = jnp.maximum(
        jnp.dot(x, W1_ref[...], preferred_element_type=jnp.float32) + b1_ref[...], 0.0)
    h = jnp.maximum(
        jnp.dot(h, W2_ref[...], preferred_element_type=jnp.float32) + b2_ref[...], 0.0)
    z = jnp.dot(h, Wfc_ref[...], preferred_element_type=jnp.float32) + bfc_ref[...]

    code_refs = (c0_ref, c1_ref, c2_ref, c3_ref)
    residual = z
    qout = jnp.zeros_like(z)
    for g in range(G):
        rn = jnp.sum(residual * residual, axis=-1, keepdims=True)
        zc = jax.lax.dot_general(residual.astype(jnp.bfloat16), cbh_ref[g],
                                 (((1,), (1,)), ((), ())),
                                 preferred_element_type=jnp.float32)
        cb32 = cb_ref[g]
        cn = jnp.sum(cb32 * cb32, axis=-1)[None, :]
        dist = (rn - 2.0 * zc) + cn
        m = jnp.min(dist, axis=-1, keepdims=True)
        iota = jax.lax.broadcasted_iota(jnp.int32, dist.shape, 1)
        idx = jnp.min(jnp.where(dist == m, iota, K), axis=-1)
        oh = (iota == idx[:, None]).astype(jnp.float32)
        q = jax.lax.dot_general(oh, cb32, (((1,), (0,)), ((), ())),
                                preferred_element_type=jnp.float32,
                                precision=jax.lax.Precision.HIGHEST)
        residual = residual - q
        qout = qout + q
        code_refs[g][...] = idx
    out_ref[...] = z + (qout - z)


def kernel(state, W1, b1, W2, b2, Wfc, bfc, codebooks):
    B = state.shape[0]
    x = state.reshape(B, -1)
    in_dim = x.shape[1]
    D = Wfc.shape[1]
    G, K, _ = codebooks.shape

    cbh = codebooks.astype(jnp.bfloat16)

    bm = min(1024, B)
    grid = (B // bm,)

    full = lambda shape: pl.BlockSpec(shape, lambda i: tuple(0 for _ in shape))
    out_vq, c0, c1, c2, c3 = pl.pallas_call(
        functools.partial(_vq_body, G=G, K=K),
        grid=grid,
        in_specs=[
            pl.BlockSpec((bm, in_dim), lambda i: (i, 0)),
            full(W1.shape), full(b1.shape), full(W2.shape), full(b2.shape),
            full(Wfc.shape), full(bfc.shape),
            full(codebooks.shape), full(cbh.shape),
        ],
        out_specs=[
            pl.BlockSpec((bm, D), lambda i: (i, 0)),
            pl.BlockSpec((bm,), lambda i: (i,)),
            pl.BlockSpec((bm,), lambda i: (i,)),
            pl.BlockSpec((bm,), lambda i: (i,)),
            pl.BlockSpec((bm,), lambda i: (i,)),
        ],
        out_shape=[
            jax.ShapeDtypeStruct((B, D), jnp.float32),
            jax.ShapeDtypeStruct((B,), jnp.int32),
            jax.ShapeDtypeStruct((B,), jnp.int32),
            jax.ShapeDtypeStruct((B,), jnp.int32),
            jax.ShapeDtypeStruct((B,), jnp.int32),
        ],
    )(x, W1, b1, W2, b2, Wfc, bfc, codebooks, cbh)
    vq_code = jnp.stack([c0, c1, c2, c3], axis=-1)
    return out_vq, vq_code
